# baseline (speedup 1.0000x reference)
import functools

import jax
import jax.numpy as jnp
from jax import lax
from jax.experimental import pallas as pl
from jax.experimental.pallas import tpu as pltpu
from jax.experimental.pallas import tpu_sc as plsc

N = 320000
D = 128
S = 1024

NC = 2
NS = 16
NW = NC * NS
RPW = N // NW
IPS = 100
GPW = RPW // IPS
NG = N // IPS
CW = 16

NCH_A = GPW
NCH_C = GPW

_MESH = plsc.VectorSubcoreMesh(
    core_axis_name="c", subcore_axis_name="s", num_cores=NC, num_subcores=NS
)


@functools.partial(
    pl.kernel,
    out_type=(
        jax.ShapeDtypeStruct((NC, S, D), jnp.float32),
        jax.ShapeDtypeStruct((NC, S, D), jnp.float32),
    ),
    mesh=_MESH,
    scratch_types=[
        pltpu.VMEM((GPW, IPS), jnp.int32),
        pltpu.VMEM((2, IPS, D), jnp.float32),
        pltpu.VMEM((IPS, D), jnp.float32),
        pltpu.VMEM_SHARED((S, D), jnp.float32),
        pltpu.VMEM_SHARED((S, D), jnp.float32),
        pltpu.SemaphoreType.DMA,
        pltpu.SemaphoreType.DMA,
        pltpu.SemaphoreType.DMA,
        pltpu.SemaphoreType.DMA,
        pltpu.SemaphoreType.DMA,
    ],
)
def _segsum_kernel(hv3_hbm, bid3_hbm,
                   psum_hbm, pcnt_hbm,
                   idx_v, chunk_v, ones_v, acc_s, cnt_s,
                   semh0, semh1, sema0, sema1, semc):
    semh = (semh0, semh1)
    sema = (sema0, sema1)
    c = lax.axis_index("c")
    s = lax.axis_index("s")
    w = s * NC + c
    rows = S // NS

    def h_copy(i, b):
        return pltpu.make_async_copy(
            hv3_hbm.at[w * GPW + i], chunk_v.at[b], semh[b])

    def a_copy(i, b):
        return pltpu.make_async_copy(
            chunk_v.at[b], acc_s.at[idx_v.at[i]], sema[b])

    zv = jnp.zeros((16,), jnp.float32)

    def zero_body(r, carry):
        for j in range(D // 16):
            chunk_v[0, r, pl.ds(j * 16, 16)] = zv
        return carry

    lax.fori_loop(0, rows, zero_body, 0)
    pltpu.sync_copy(chunk_v.at[0].at[pl.ds(0, rows)],
                    acc_s.at[pl.ds(s * rows, rows)])
    pltpu.sync_copy(chunk_v.at[0].at[pl.ds(0, rows)],
                    cnt_s.at[pl.ds(s * rows, rows)])
    ov = jnp.ones((16,), jnp.float32)

    def ones_body(r, carry):
        for j in range(D // 16):
            ones_v[r, pl.ds(j * 16, 16)] = ov
        return carry

    lax.fori_loop(0, IPS, ones_body, 0)
    pltpu.sync_copy(bid3_hbm.at[w], idx_v)
    plsc.subcore_barrier()
    h_copy(0, 0).start()

    def pair_body(k, carry):
        for b in range(2):
            i = 2 * k + b
            nb = 1 - b

            @pl.when(i > 0)
            def _wait_prev_scatter():
                a_copy(i - 1, nb).wait()

            @pl.when(i < NCH_A - 1)
            def _start_next_load():
                h_copy(i + 1, nb).start()

            h_copy(i, b).wait()
            pltpu.async_copy(chunk_v.at[b], acc_s.at[idx_v.at[i]],
                             sema[b], add=True)
            pltpu.async_copy(ones_v, cnt_s.at[idx_v.at[i]], semc, add=True)
        return carry

    lax.fori_loop(0, NCH_A // 2, pair_body, 0)
    a_copy(NCH_A - 1, 1).wait()

    def cnt_drain(i, carry):
        pltpu.make_async_copy(ones_v, cnt_s.at[idx_v.at[i]], semc).wait()
        return carry

    lax.fori_loop(0, GPW, cnt_drain, 0)
    plsc.subcore_barrier()

    pltpu.sync_copy(acc_s.at[pl.ds(s * rows, rows)],
                    psum_hbm.at[c].at[pl.ds(s * rows, rows)])
    pltpu.sync_copy(cnt_s.at[pl.ds(s * rows, rows)],
                    pcnt_hbm.at[c].at[pl.ds(s * rows, rows)])


@functools.partial(
    pl.kernel,
    out_type=jax.ShapeDtypeStruct((NG, IPS, D), jnp.float32),
    mesh=_MESH,
    scratch_types=[
        pltpu.VMEM((GPW, IPS), jnp.int32),
        pltpu.VMEM((2, IPS, D), jnp.float32),
        pltpu.VMEM((2, IPS, D), jnp.float32),
        pltpu.VMEM((2, D, D), jnp.float32),
        pltpu.VMEM((2, 1, D), jnp.float32),
        pltpu.VMEM_SHARED((S, D), jnp.float32),
        pltpu.SemaphoreType.DMA,
        pltpu.SemaphoreType.DMA,
        pltpu.SemaphoreType.DMA,
        pltpu.SemaphoreType.DMA,
        pltpu.SemaphoreType.DMA,
        pltpu.SemaphoreType.DMA,
    ],
)
def _gatemul_kernel(hv3_hbm, bid3_hbm, psum_hbm, pcnt_hbm,
                    w1_hbm, b1_hbm, w2_hbm, b2_hbm, out3_hbm,
                    idx_v, h_v, g_v, w_v, b_v, gate_s,
                    semh0, semh1, semg0, semg1, semo0, semo1):
    semh = (semh0, semh1)
    semg = (semg0, semg1)
    semo = (semo0, semo1)
    c = lax.axis_index("c")
    s = lax.axis_index("s")
    w = s * NC + c
    rows = S // NS
    RB = 4
    NSL = D // 16

    def h_copy(i, b):
        return pltpu.make_async_copy(
            hv3_hbm.at[w * GPW + i], h_v.at[b], semh[b])

    def g_copy(i, b):
        return pltpu.make_async_copy(
            gate_s.at[idx_v.at[i]], g_v.at[b], semg[b])

    def o_copy(i, b):
        return pltpu.make_async_copy(
            h_v.at[b], out3_hbm.at[w * GPW + i], semo[b])

    pltpu.sync_copy(bid3_hbm.at[w], idx_v)
    pltpu.sync_copy(w1_hbm, w_v.at[0])
    pltpu.sync_copy(w2_hbm, w_v.at[1])
    pltpu.sync_copy(b1_hbm, b_v.at[0])
    pltpu.sync_copy(b2_hbm, b_v.at[1])
    pltpu.sync_copy(psum_hbm.at[0].at[pl.ds(s * rows, rows)],
                    h_v.at[0].at[pl.ds(0, rows)])
    pltpu.sync_copy(psum_hbm.at[1].at[pl.ds(s * rows, rows)],
                    h_v.at[1].at[pl.ds(0, rows)])
    pltpu.sync_copy(pcnt_hbm.at[0].at[pl.ds(s * rows, rows)],
                    g_v.at[0].at[pl.ds(0, rows)])
    pltpu.sync_copy(pcnt_hbm.at[1].at[pl.ds(s * rows, rows)],
                    g_v.at[1].at[pl.ds(0, rows)])

    def mean_body(r, carry):
        for j in range(NSL):
            sl = pl.ds(j * 16, 16)
            cnt = jnp.maximum(g_v[0, r, sl] + g_v[1, r, sl], 1.0)
            h_v[0, r, sl] = (h_v[0, r, sl] + h_v[1, r, sl]) / cnt
        return carry

    lax.fori_loop(0, rows, mean_body, 0)

    def layer(src_ref, dst_sel, wsel, act):
        def blk_body(blk, carry):
            r0 = blk * RB

            def k_body(kb, accs):
                k0 = kb * 16
                cvecs = [src_ref[r0 + r, pl.ds(k0, 16)] for r in range(RB)]
                for kk in range(16):
                    wvecs = [w_v[wsel, k0 + kk, pl.ds(j * 16, 16)]
                             for j in range(NSL)]
                    accs = tuple(tuple(accs[r][j] + cvecs[r][kk] * wvecs[j]
                                       for j in range(NSL))
                                 for r in range(RB))
                return accs

            zero = jnp.zeros((16,), jnp.float32)
            init = tuple(tuple(zero for _ in range(NSL)) for _ in range(RB))
            accs = lax.fori_loop(0, D // 16, k_body, init)
            for r in range(RB):
                for j in range(NSL):
                    sl = pl.ds(j * 16, 16)
                    dst_sel(r0 + r, sl, act(accs[r][j] + b_v[wsel, 0, sl]))
            return carry

        lax.fori_loop(0, rows // RB, blk_body, 0)

    def relu(x):
        return jnp.maximum(x, 0.0)

    def sigm(x):
        return 1.0 / (1.0 + jnp.exp(-x))

    def store_g0(r, sl, val):
        g_v[0, r, sl] = val

    def store_h1(r, sl, val):
        h_v[1, r, sl] = val

    layer(h_v.at[0], store_g0, 0, relu)
    layer(g_v.at[0], store_h1, 1, sigm)

    pltpu.sync_copy(h_v.at[1].at[pl.ds(0, rows)],
                    gate_s.at[pl.ds(s * rows, rows)])
    h_copy(0, 0).start()
    plsc.subcore_barrier()
    g_copy(0, 0).start()

    def pair_body(k, carry):
        for b in range(2):
            i = 2 * k + b
            nb = 1 - b

            @pl.when(i > 0)
            def _wait_prev_out():
                o_copy(i - 1, nb).wait()

            @pl.when(i < NCH_C - 1)
            def _start_next_loads():
                h_copy(i + 1, nb).start()
                g_copy(i + 1, nb).start()

            h_copy(i, b).wait()
            g_copy(i, b).wait()

            def row_body(r, carry2):
                for j in range(D // 16):
                    sl = pl.ds(j * 16, 16)
                    h_v[b, r, sl] = h_v[b, r, sl] * g_v[b, r, sl]
                return carry2

            lax.fori_loop(0, IPS, row_body, 0)
            o_copy(i, b).start()
        return carry

    lax.fori_loop(0, NCH_C // 2, pair_body, 0)
    o_copy(NCH_C - 1, 1).wait()


def kernel(h_V, batch_id, W1, b1, W2, b2):
    hv3 = h_V.reshape(NG, IPS, D)
    bid3 = batch_id.astype(jnp.int32).reshape(NW, GPW, IPS)
    psum, pcnt = _segsum_kernel(hv3, bid3)
    out3 = _gatemul_kernel(hv3, bid3, psum, pcnt,
                           W1, b1.reshape(1, D), W2, b2.reshape(1, D))
    return out3.reshape(N, D)


# speedup vs baseline: 2.0268x; 2.0268x over previous
import functools

import jax
import jax.numpy as jnp
from jax import lax
from jax.experimental import pallas as pl
from jax.experimental.pallas import tpu as pltpu
from jax.experimental.pallas import tpu_sc as plsc

N = 320000
D = 128
S = 1024

NC = 2
NS = 16
NW = NC * NS
RPW = N // NW
IPS = 80
GPW = RPW // IPS
NG = N // IPS
CW = 16

NCH_A = GPW
NCH_C = GPW

_MESH = plsc.VectorSubcoreMesh(
    core_axis_name="c", subcore_axis_name="s", num_cores=NC, num_subcores=NS
)


@functools.partial(
    pl.kernel,
    out_type=(
        jax.ShapeDtypeStruct((NC, S, D), jnp.float32),
        jax.ShapeDtypeStruct((NC, S, D), jnp.float32),
    ),
    mesh=_MESH,
    scratch_types=[
        pltpu.VMEM((GPW, IPS), jnp.int32),
        pltpu.VMEM((2, IPS, D), jnp.float32),
        pltpu.VMEM((IPS, D), jnp.float32),
        pltpu.VMEM_SHARED((S, D), jnp.float32),
        pltpu.VMEM_SHARED((S, D), jnp.float32),
        pltpu.SemaphoreType.DMA,
        pltpu.SemaphoreType.DMA,
        pltpu.SemaphoreType.DMA,
    ],
)
def _segsum_kernel(hv3_hbm, bid3_hbm,
                   psum_hbm, pcnt_hbm,
                   idx_v, chunk_v, ones_v, acc_s, cnt_s,
                   semh0, semh1, semc):
    semh = (semh0, semh1)
    c = lax.axis_index("c")
    s = lax.axis_index("s")
    w = s * NC + c
    rows = S // NS

    def h_copy(i, b):
        return pltpu.make_async_copy(
            hv3_hbm.at[w * GPW + i], chunk_v.at[b], semh[b])

    zv = jnp.zeros((16,), jnp.float32)

    def zero_body(r, carry):
        for j in range(D // 16):
            chunk_v[0, r, pl.ds(j * 16, 16)] = zv
        return carry

    lax.fori_loop(0, rows, zero_body, 0)
    pltpu.sync_copy(chunk_v.at[0].at[pl.ds(0, rows)],
                    acc_s.at[pl.ds(s * rows, rows)])
    pltpu.sync_copy(chunk_v.at[0].at[pl.ds(0, rows)],
                    cnt_s.at[pl.ds(s * rows, rows)])
    ov = jnp.ones((16,), jnp.float32)

    def ones_body(r, carry):
        for j in range(D // 16):
            ones_v[r, pl.ds(j * 16, 16)] = ov
        return carry

    lax.fori_loop(0, IPS, ones_body, 0)
    pltpu.sync_copy(bid3_hbm.at[w], idx_v)
    plsc.subcore_barrier()
    h_copy(0, 0).start()

    def pair_body(k, carry):
        for b in range(2):
            i = 2 * k + b
            nb = 1 - b

            @pl.when(i < NCH_A - 1)
            def _start_next_load():
                h_copy(i + 1, nb).start()

            h_copy(i, b).wait()
            pltpu.sync_copy(chunk_v.at[b], acc_s.at[idx_v.at[i]], add=True)
            pltpu.async_copy(ones_v, cnt_s.at[idx_v.at[i]], semc, add=True)
        return carry

    lax.fori_loop(0, NCH_A // 2, pair_body, 0)
    h_copy(NCH_A - 1, 0).wait()
    pltpu.sync_copy(chunk_v.at[0], acc_s.at[idx_v.at[NCH_A - 1]], add=True)
    pltpu.async_copy(ones_v, cnt_s.at[idx_v.at[NCH_A - 1]], semc, add=True)

    def cnt_drain(i, carry):
        pltpu.make_async_copy(ones_v, cnt_s.at[idx_v.at[i]], semc).wait()
        return carry

    lax.fori_loop(0, GPW, cnt_drain, 0)
    plsc.subcore_barrier()

    pltpu.sync_copy(acc_s.at[pl.ds(s * rows, rows)],
                    psum_hbm.at[c].at[pl.ds(s * rows, rows)])
    pltpu.sync_copy(cnt_s.at[pl.ds(s * rows, rows)],
                    pcnt_hbm.at[c].at[pl.ds(s * rows, rows)])


def _mlp_body(psum_ref, pcnt_ref, w1_ref, b1_ref, w2_ref, b2_ref, gate_ref):
    sums = psum_ref[0] + psum_ref[1]
    cnt_rows = pcnt_ref[0] + pcnt_ref[1]
    counts = jnp.sum(cnt_rows, axis=1) * (1.0 / D)
    c_v = sums / jnp.clip(counts, 1.0, None)[:, None]
    hmid = jnp.maximum(
        jnp.dot(c_v, w1_ref[...], preferred_element_type=jnp.float32)
        + b1_ref[...], 0.0)
    logits = (jnp.dot(hmid, w2_ref[...], preferred_element_type=jnp.float32)
              + b2_ref[...])
    gate_ref[...] = 1.0 / (1.0 + jnp.exp(-logits))


_mlp_call = pl.pallas_call(
    _mlp_body,
    out_shape=jax.ShapeDtypeStruct((S, D), jnp.float32),
)


@functools.partial(
    pl.kernel,
    out_type=jax.ShapeDtypeStruct((NG, IPS, D), jnp.float32),
    mesh=_MESH,
    scratch_types=[
        pltpu.VMEM((GPW, IPS), jnp.int32),
        pltpu.VMEM((2, IPS, D), jnp.float32),
        pltpu.VMEM((2, IPS, D), jnp.float32),
        pltpu.VMEM_SHARED((S, D), jnp.float32),
        pltpu.SemaphoreType.DMA,
        pltpu.SemaphoreType.DMA,
        pltpu.SemaphoreType.DMA,
        pltpu.SemaphoreType.DMA,
        pltpu.SemaphoreType.DMA,
        pltpu.SemaphoreType.DMA,
    ],
)
def _gatemul_kernel(hv3_hbm, bid3_hbm, gate_hbm, out3_hbm,
                    idx_v, h_v, g_v, gate_s,
                    semh0, semh1, semg0, semg1, semo0, semo1):
    semh = (semh0, semh1)
    semg = (semg0, semg1)
    semo = (semo0, semo1)
    c = lax.axis_index("c")
    s = lax.axis_index("s")
    w = s * NC + c

    def h_copy(i, b):
        return pltpu.make_async_copy(
            hv3_hbm.at[w * GPW + i], h_v.at[b], semh[b])

    def g_copy(i, b):
        return pltpu.make_async_copy(
            gate_s.at[idx_v.at[i]], g_v.at[b], semg[b])

    def o_copy(i, b):
        return pltpu.make_async_copy(
            h_v.at[b], out3_hbm.at[w * GPW + i], semo[b])

    h_copy(0, 0).start()
    pltpu.sync_copy(bid3_hbm.at[w], idx_v)
    rows = S // NS
    pltpu.sync_copy(gate_hbm.at[pl.ds(s * rows, rows)],
                    gate_s.at[pl.ds(s * rows, rows)])
    plsc.subcore_barrier()
    g_copy(0, 0).start()

    def pair_body(k, carry):
        for b in range(2):
            i = 2 * k + b
            nb = 1 - b

            @pl.when(i > 0)
            def _wait_prev_out():
                o_copy(i - 1, nb).wait()

            @pl.when(i < NCH_C - 1)
            def _start_next_loads():
                h_copy(i + 1, nb).start()
                g_copy(i + 1, nb).start()

            h_copy(i, b).wait()
            g_copy(i, b).wait()

            def row_body(r, carry2):
                for j in range(D // 16):
                    sl = pl.ds(j * 16, 16)
                    h_v[b, r, sl] = h_v[b, r, sl] * g_v[b, r, sl]
                return carry2

            lax.fori_loop(0, IPS, row_body, 0)
            o_copy(i, b).start()
        return carry

    lax.fori_loop(0, NCH_C // 2, pair_body, 0)
    o_copy(NCH_C - 2, 1).wait()
    h_copy(NCH_C - 1, 0).wait()
    g_copy(NCH_C - 1, 0).wait()

    def last_row_body(r, carry2):
        for j in range(D // 16):
            sl = pl.ds(j * 16, 16)
            h_v[0, r, sl] = h_v[0, r, sl] * g_v[0, r, sl]
        return carry2

    lax.fori_loop(0, IPS, last_row_body, 0)
    o_copy(NCH_C - 1, 0).start()
    o_copy(NCH_C - 1, 0).wait()


def kernel(h_V, batch_id, W1, b1, W2, b2):
    hv3 = h_V.reshape(NG, IPS, D)
    bid3 = batch_id.astype(jnp.int32).reshape(NW, GPW, IPS)
    psum, pcnt = _segsum_kernel(hv3, bid3)
    gate = _mlp_call(psum, pcnt, W1, b1.reshape(1, D), W2, b2.reshape(1, D))
    out3 = _gatemul_kernel(hv3, bid3, gate)
    return out3.reshape(N, D)
